# baseline (device time: 152138 ns/iter reference)
import jax
import jax.numpy as jnp
from jax import lax
from jax.experimental import pallas as pl
from jax.experimental.pallas import tpu as pltpu

N_DEV = 4


def kernel(x, w_mat):
    m, k_per = x.shape
    _, n = w_mat.shape
    m_out = m // N_DEV

    def body(x_ref, w_ref, out_ref, comm_ref, send_sems, recv_sems):
        my = lax.axis_index("i")
        left = lax.rem(my + N_DEV - 1, N_DEV)
        right = lax.rem(my + 1, N_DEV)

        barrier_sem = pltpu.get_barrier_semaphore()
        for nbr in (left, right):
            pl.semaphore_signal(
                barrier_sem, inc=1,
                device_id=(nbr,), device_id_type=pl.DeviceIdType.MESH,
            )
        pl.semaphore_wait(barrier_sem, 2)

        def chunk_partial(c):
            xs = x_ref[pl.ds(c * m_out, m_out), :]
            return jnp.dot(xs, w_ref[:, :], preferred_element_type=jnp.float32)

        comm_ref[0, :, :] = chunk_partial(left)

        for s in range(N_DEV - 1):
            send_slot = s % 2
            recv_slot = (s + 1) % 2
            rdma = pltpu.make_async_remote_copy(
                src_ref=comm_ref.at[send_slot],
                dst_ref=comm_ref.at[recv_slot],
                send_sem=send_sems.at[send_slot],
                recv_sem=recv_sems.at[recv_slot],
                device_id=(right,),
                device_id_type=pl.DeviceIdType.MESH,
            )
            rdma.start()
            c_recv = lax.rem(my + 2 * N_DEV - 2 - s, N_DEV)
            p = chunk_partial(c_recv)
            rdma.wait()
            if s < N_DEV - 2:
                comm_ref[recv_slot, :, :] = comm_ref[recv_slot, :, :] + p
            else:
                out_ref[:, :] = jnp.maximum(comm_ref[recv_slot, :, :] + p, 0.0)

    return pl.pallas_call(
        body,
        out_shape=jax.ShapeDtypeStruct((m_out, n), jnp.float32),
        in_specs=[
            pl.BlockSpec(memory_space=pltpu.VMEM),
            pl.BlockSpec(memory_space=pltpu.VMEM),
        ],
        out_specs=pl.BlockSpec(memory_space=pltpu.VMEM),
        scratch_shapes=[
            pltpu.VMEM((2, m_out, n), jnp.float32),
            pltpu.SemaphoreType.DMA((2,)),
            pltpu.SemaphoreType.DMA((2,)),
        ],
        compiler_params=pltpu.CompilerParams(collective_id=0),
    )(x, w_mat)


# device time: 84793 ns/iter; 1.7942x vs baseline; 1.7942x over previous
import jax
import jax.numpy as jnp
from jax import lax
from jax.experimental import pallas as pl
from jax.experimental.pallas import tpu as pltpu

N_DEV = 4


def kernel(x, w_mat):
    m, k_per = x.shape
    _, n = w_mat.shape
    m_out = m // N_DEV
    nh = n // 2

    def body(x_ref, w_ref, out_ref, comm_r, comm_l,
             send_sems_r, recv_sems_r, send_sems_l, recv_sems_l):
        my = lax.axis_index("i")
        left = lax.rem(my + N_DEV - 1, N_DEV)
        right = lax.rem(my + 1, N_DEV)

        barrier_sem = pltpu.get_barrier_semaphore()
        for nbr in (left, right):
            pl.semaphore_signal(
                barrier_sem, inc=1,
                device_id=(nbr,), device_id_type=pl.DeviceIdType.MESH,
            )
        pl.semaphore_wait(barrier_sem, 2)

        def partial_half(c, col0):
            xs = x_ref[pl.ds(c * m_out, m_out), :]
            return jnp.dot(xs, w_ref[:, col0:col0 + nh],
                           preferred_element_type=jnp.float32)

        comm_r[0, :, :] = partial_half(left, 0)
        comm_l[0, :, :] = partial_half(right, nh)

        for s in range(N_DEV - 1):
            send_slot = s % 2
            recv_slot = (s + 1) % 2
            rdma_r = pltpu.make_async_remote_copy(
                src_ref=comm_r.at[send_slot],
                dst_ref=comm_r.at[recv_slot],
                send_sem=send_sems_r.at[send_slot],
                recv_sem=recv_sems_r.at[recv_slot],
                device_id=(right,),
                device_id_type=pl.DeviceIdType.MESH,
            )
            rdma_l = pltpu.make_async_remote_copy(
                src_ref=comm_l.at[send_slot],
                dst_ref=comm_l.at[recv_slot],
                send_sem=send_sems_l.at[send_slot],
                recv_sem=recv_sems_l.at[recv_slot],
                device_id=(left,),
                device_id_type=pl.DeviceIdType.MESH,
            )
            rdma_r.start()
            rdma_l.start()
            c_recv_r = lax.rem(my + 2 * N_DEV - 2 - s, N_DEV)
            c_recv_l = lax.rem(my + 2 + s, N_DEV)
            p_r = partial_half(c_recv_r, 0)
            p_l = partial_half(c_recv_l, nh)
            rdma_r.wait()
            rdma_l.wait()
            if s < N_DEV - 2:
                comm_r[recv_slot, :, :] = comm_r[recv_slot, :, :] + p_r
                comm_l[recv_slot, :, :] = comm_l[recv_slot, :, :] + p_l
            else:
                out_ref[:, 0:nh] = jnp.maximum(comm_r[recv_slot, :, :] + p_r, 0.0)
                out_ref[:, nh:n] = jnp.maximum(comm_l[recv_slot, :, :] + p_l, 0.0)

    return pl.pallas_call(
        body,
        out_shape=jax.ShapeDtypeStruct((m_out, n), jnp.float32),
        in_specs=[
            pl.BlockSpec(memory_space=pltpu.VMEM),
            pl.BlockSpec(memory_space=pltpu.VMEM),
        ],
        out_specs=pl.BlockSpec(memory_space=pltpu.VMEM),
        scratch_shapes=[
            pltpu.VMEM((2, m_out, nh), jnp.float32),
            pltpu.VMEM((2, m_out, nh), jnp.float32),
            pltpu.SemaphoreType.DMA((2,)),
            pltpu.SemaphoreType.DMA((2,)),
            pltpu.SemaphoreType.DMA((2,)),
            pltpu.SemaphoreType.DMA((2,)),
        ],
        compiler_params=pltpu.CompilerParams(collective_id=0),
    )(x, w_mat)


# device time: 80594 ns/iter; 1.8877x vs baseline; 1.0521x over previous
import jax
import jax.numpy as jnp
from jax import lax
from jax.experimental import pallas as pl
from jax.experimental.pallas import tpu as pltpu

N_DEV = 4
P = 4


def kernel(x, w_mat):
    m, k_per = x.shape
    _, n = w_mat.shape
    m_out = m // N_DEV
    nh = n // 2
    rp = m_out // P

    def body(x_ref, w_ref, out_ref, comm_r, comm_l,
             send_sems_r, recv_sems_r, send_sems_l, recv_sems_l):
        my = lax.axis_index("i")
        left = lax.rem(my + N_DEV - 1, N_DEV)
        right = lax.rem(my + 1, N_DEV)

        barrier_sem = pltpu.get_barrier_semaphore()
        for nbr in (left, right):
            pl.semaphore_signal(
                barrier_sem, inc=1,
                device_id=(nbr,), device_id_type=pl.DeviceIdType.MESH,
            )
        pl.semaphore_wait(barrier_sem, 2)

        def partial(c, col0, w_cols):
            xs = x_ref[pl.ds(c * m_out, m_out), :]
            return jnp.dot(xs, w_ref[:, col0:col0 + w_cols],
                           preferred_element_type=jnp.float32)

        def mk(comm, ssems, rsems, h, j, nbr):
            return pltpu.make_async_remote_copy(
                src_ref=comm.at[h, pl.ds(j * rp, rp), :],
                dst_ref=comm.at[h + 1, pl.ds(j * rp, rp), :],
                send_sem=ssems.at[h, j],
                recv_sem=rsems.at[h, j],
                device_id=(nbr,),
                device_id_type=pl.DeviceIdType.MESH,
            )

        rd = {}
        for h in range(N_DEV - 1):
            for j in range(P):
                rd["r", h, j] = mk(comm_r, send_sems_r, recv_sems_r, h, j, right)
                rd["l", h, j] = mk(comm_l, send_sems_l, recv_sems_l, h, j, left)

        comm_r[0, :, :] = partial(left, 0, nh)
        comm_l[0, :, :] = partial(right, nh, nh)

        for j in range(P):
            rd["r", 0, j].start()
            rd["l", 0, j].start()

        p0 = partial(lax.rem(my + 2, N_DEV), 0, n)
        for j in range(P):
            r0, r1 = j * rp, (j + 1) * rp
            rd["r", 0, j].wait_recv()
            comm_r[1, r0:r1, :] = comm_r[1, r0:r1, :] + p0[r0:r1, 0:nh]
            rd["r", 1, j].start()
            rd["l", 0, j].wait_recv()
            comm_l[1, r0:r1, :] = comm_l[1, r0:r1, :] + p0[r0:r1, nh:n]
            rd["l", 1, j].start()

        p1r = partial(lax.rem(my + 1, N_DEV), 0, nh)
        p1l = partial(lax.rem(my + 3, N_DEV), nh, nh)
        for j in range(P):
            r0, r1 = j * rp, (j + 1) * rp
            rd["r", 1, j].wait_recv()
            comm_r[2, r0:r1, :] = comm_r[2, r0:r1, :] + p1r[r0:r1, :]
            rd["r", 2, j].start()
            rd["l", 1, j].wait_recv()
            comm_l[2, r0:r1, :] = comm_l[2, r0:r1, :] + p1l[r0:r1, :]
            rd["l", 2, j].start()

        p2 = partial(my, 0, n)
        for j in range(P):
            r0, r1 = j * rp, (j + 1) * rp
            rd["r", 2, j].wait_recv()
            out_ref[r0:r1, 0:nh] = jnp.maximum(
                comm_r[3, r0:r1, :] + p2[r0:r1, 0:nh], 0.0)
            rd["l", 2, j].wait_recv()
            out_ref[r0:r1, nh:n] = jnp.maximum(
                comm_l[3, r0:r1, :] + p2[r0:r1, nh:n], 0.0)

        for h in range(N_DEV - 1):
            for j in range(P):
                rd["r", h, j].wait_send()
                rd["l", h, j].wait_send()

    return pl.pallas_call(
        body,
        out_shape=jax.ShapeDtypeStruct((m_out, n), jnp.float32),
        in_specs=[
            pl.BlockSpec(memory_space=pltpu.VMEM),
            pl.BlockSpec(memory_space=pltpu.VMEM),
        ],
        out_specs=pl.BlockSpec(memory_space=pltpu.VMEM),
        scratch_shapes=[
            pltpu.VMEM((N_DEV, m_out, nh), jnp.float32),
            pltpu.VMEM((N_DEV, m_out, nh), jnp.float32),
            pltpu.SemaphoreType.DMA((N_DEV - 1, P)),
            pltpu.SemaphoreType.DMA((N_DEV - 1, P)),
            pltpu.SemaphoreType.DMA((N_DEV - 1, P)),
            pltpu.SemaphoreType.DMA((N_DEV - 1, P)),
        ],
        compiler_params=pltpu.CompilerParams(collective_id=0),
    )(x, w_mat)
